# Initial kernel scaffold; baseline (speedup 1.0000x reference)
import jax
import jax.numpy as jnp
from jax.experimental import pallas as pl


def kernel(x, index):
    raise NotImplementedError("write your pallas kernel here")


# speedup vs baseline: 3.7938x; 3.7938x over previous
import functools

import jax
import jax.numpy as jnp
from jax import lax
from jax.experimental import pallas as pl
from jax.experimental.pallas import tpu as pltpu
from jax.experimental.pallas import tpu_sc as plsc

N_NODES_K = 10000
N_EDGES_K = 320000
D_FEAT_K = 128
SCALE = 1.0 / (32.0 ** 0.5)

NUM_CORES = 2
NUM_SUBCORES = 16
EDGES_PER_CORE = N_EDGES_K // NUM_CORES
EDGES_PER_TILE = EDGES_PER_CORE // NUM_SUBCORES
WINDOW = 80
N_WINDOWS = EDGES_PER_TILE // WINDOW
ROWS_PER_TILE = 624
ROWS_TAIL = N_NODES_K - NUM_SUBCORES * ROWS_PER_TILE
TAIL_ROW0 = NUM_SUBCORES * ROWS_PER_TILE


def _sc_scatter_partials(x, index):
    mesh = plsc.VectorSubcoreMesh(core_axis_name="c", subcore_axis_name="s")

    @functools.partial(
        pl.kernel,
        out_type=jax.ShapeDtypeStruct((NUM_CORES, N_NODES_K, D_FEAT_K),
                                      jnp.float32),
        mesh=mesh,
        scratch_types=[
            pltpu.VMEM_SHARED((N_NODES_K, D_FEAT_K), jnp.float32),
            pltpu.VMEM((WINDOW,), jnp.int32),
            pltpu.VMEM((WINDOW, D_FEAT_K), jnp.float32),
        ],
    )
    def k(x_hbm, idx_hbm, out_hbm, acc_sp, idx_v, x_v):
        c = lax.axis_index("c")
        s = lax.axis_index("s")

        @pl.loop(0, WINDOW)
        def _(i):
            @pl.loop(0, D_FEAT_K, step=16)
            def _(j):
                x_v.at[i, pl.ds(j, 16)][...] = jnp.zeros((16,), jnp.float32)

        row0 = s * ROWS_PER_TILE

        @pl.loop(0, ROWS_PER_TILE // WINDOW)
        def _(i):
            pltpu.sync_copy(x_v, acc_sp.at[pl.ds(row0 + i * WINDOW, WINDOW)])

        rem = ROWS_PER_TILE - (ROWS_PER_TILE // WINDOW) * WINDOW
        if rem:
            pltpu.sync_copy(x_v.at[pl.ds(0, rem)],
                            acc_sp.at[pl.ds(row0 + ROWS_PER_TILE - rem, rem)])

        @pl.when(s == 0)
        def _():
            pltpu.sync_copy(x_v.at[pl.ds(0, ROWS_TAIL)],
                            acc_sp.at[pl.ds(TAIL_ROW0, ROWS_TAIL)])

        plsc.subcore_barrier()

        base_tile = c * EDGES_PER_CORE + s * EDGES_PER_TILE

        @pl.loop(0, N_WINDOWS)
        def _(w):
            base = base_tile + w * WINDOW
            pltpu.sync_copy(idx_hbm.at[pl.ds(base, WINDOW)], idx_v)
            pltpu.sync_copy(x_hbm.at[pl.ds(base, WINDOW)], x_v)
            pltpu.sync_copy(x_v, acc_sp.at[idx_v], add=True)

        plsc.subcore_barrier()
        pltpu.sync_copy(acc_sp.at[pl.ds(row0, ROWS_PER_TILE)],
                        out_hbm.at[c, pl.ds(row0, ROWS_PER_TILE)])

        @pl.when(s == 0)
        def _():
            pltpu.sync_copy(acc_sp.at[pl.ds(TAIL_ROW0, ROWS_TAIL)],
                            out_hbm.at[c, pl.ds(TAIL_ROW0, ROWS_TAIL)])

    return k(x, index)


def _tc_combine_body(p_ref, o_ref):
    o_ref[...] = (p_ref[0] + p_ref[1]) * SCALE


def _tc_combine(partials):
    blk = 1000
    return pl.pallas_call(
        _tc_combine_body,
        grid=(N_NODES_K // blk,),
        in_specs=[pl.BlockSpec((NUM_CORES, blk, D_FEAT_K),
                               lambda i: (0, i, 0))],
        out_specs=pl.BlockSpec((blk, D_FEAT_K), lambda i: (i, 0)),
        out_shape=jax.ShapeDtypeStruct((N_NODES_K, D_FEAT_K), jnp.float32),
    )(partials)


@jax.jit
def kernel(x, index):
    index = index.astype(jnp.int32)
    partials = _sc_scatter_partials(x, index)
    return _tc_combine(partials)


# speedup vs baseline: 8.3254x; 2.1945x over previous
import functools

import jax
import jax.numpy as jnp
from jax import lax
from jax.experimental import pallas as pl
from jax.experimental.pallas import tpu as pltpu
from jax.experimental.pallas import tpu_sc as plsc

N_NODES_K = 10000
N_EDGES_K = 320000
D_FEAT_K = 128
SCALE = 1.0 / (32.0 ** 0.5)

NUM_CORES = 2
NUM_SUBCORES = 16
NUM_TILES = NUM_CORES * NUM_SUBCORES
WINDOW = 128
N_WINDOWS = N_EDGES_K // WINDOW
MAX_W_PER_TILE = -(-N_WINDOWS // NUM_TILES)
NBUF = 2

ROWS_PER_TILE = 624
ROWS_TAIL = N_NODES_K - NUM_SUBCORES * ROWS_PER_TILE
TAIL_ROW0 = NUM_SUBCORES * ROWS_PER_TILE


def _sc_scatter_partials(x, idx2d):
    mesh = plsc.VectorSubcoreMesh(core_axis_name="c", subcore_axis_name="s")

    @functools.partial(
        pl.kernel,
        out_type=jax.ShapeDtypeStruct((NUM_CORES, N_NODES_K, D_FEAT_K),
                                      jnp.float32),
        mesh=mesh,
        scratch_types=[
            pltpu.VMEM_SHARED((N_NODES_K, D_FEAT_K), jnp.float32),
            pltpu.VMEM((NBUF, WINDOW), jnp.int32),
            pltpu.VMEM((WINDOW, D_FEAT_K), jnp.float32),
            pltpu.VMEM((WINDOW, D_FEAT_K), jnp.float32),
            pltpu.SemaphoreType.DMA((NBUF,)),
        ],
    )
    def k(x_hbm, idx_hbm, out_hbm, acc_sp, idx_v, x_v0, x_v1, ld_sem):
        c = lax.axis_index("c")
        s = lax.axis_index("s")
        wid = c * NUM_SUBCORES + s
        xs = [x_v0, x_v1]

        @pl.loop(0, WINDOW)
        def _(i):
            @pl.loop(0, D_FEAT_K, step=16)
            def _(j):
                x_v0.at[i, pl.ds(j, 16)][...] = jnp.zeros((16,), jnp.float32)

        row0 = s * ROWS_PER_TILE
        n_full = ROWS_PER_TILE // WINDOW
        for i in range(n_full):
            pltpu.sync_copy(x_v0, acc_sp.at[pl.ds(row0 + i * WINDOW, WINDOW)])
        rem = ROWS_PER_TILE - n_full * WINDOW
        if rem:
            pltpu.sync_copy(x_v0.at[pl.ds(0, rem)],
                            acc_sp.at[pl.ds(row0 + ROWS_PER_TILE - rem, rem)])

        @pl.when(s == 0)
        def _():
            pltpu.sync_copy(x_v0.at[pl.ds(0, ROWS_TAIL)],
                            acc_sp.at[pl.ds(TAIL_ROW0, ROWS_TAIL)])

        def g_of(w):
            return w * NUM_TILES + wid

        def start_load(w, b):
            @pl.when(g_of(w) < N_WINDOWS)
            def _():
                g = g_of(w)
                pltpu.async_copy(idx_hbm.at[g], idx_v.at[b], ld_sem.at[b])
                pltpu.async_copy(x_hbm.at[pl.ds(g * WINDOW, WINDOW)],
                                 xs[b], ld_sem.at[b])

        def wait_load(w, b):
            @pl.when(g_of(w) < N_WINDOWS)
            def _():
                pltpu.make_async_copy(idx_hbm.at[0], idx_v.at[b],
                                      ld_sem.at[b]).wait()
                pltpu.make_async_copy(x_hbm.at[pl.ds(0, WINDOW)], xs[b],
                                      ld_sem.at[b]).wait()

        def sync_scatter(w, b):
            @pl.when(g_of(w) < N_WINDOWS)
            def _():
                pltpu.sync_copy(xs[b], acc_sp.at[idx_v.at[b]], add=True)

        start_load(0, 0)
        plsc.subcore_barrier()

        n_main = (MAX_W_PER_TILE - 1) // NBUF * NBUF

        @pl.loop(0, n_main, step=NBUF)
        def _(k0):
            for b in range(NBUF):
                w = k0 + b
                nb = (b + 1) % NBUF
                start_load(w + 1, nb)
                wait_load(w, b)
                sync_scatter(w, b)

        wait_load(n_main, 0)
        sync_scatter(n_main, 0)

        plsc.subcore_barrier()
        pltpu.sync_copy(acc_sp.at[pl.ds(row0, ROWS_PER_TILE)],
                        out_hbm.at[c, pl.ds(row0, ROWS_PER_TILE)])

        @pl.when(s == 0)
        def _():
            pltpu.sync_copy(acc_sp.at[pl.ds(TAIL_ROW0, ROWS_TAIL)],
                            out_hbm.at[c, pl.ds(TAIL_ROW0, ROWS_TAIL)])

    return k(x, idx2d)


def _tc_combine_body(p_ref, o_ref):
    o_ref[...] = (p_ref[0] + p_ref[1]) * SCALE


def _tc_combine(partials):
    blk = 1000
    return pl.pallas_call(
        _tc_combine_body,
        grid=(N_NODES_K // blk,),
        in_specs=[pl.BlockSpec((NUM_CORES, blk, D_FEAT_K),
                               lambda i: (0, i, 0))],
        out_specs=pl.BlockSpec((blk, D_FEAT_K), lambda i: (i, 0)),
        out_shape=jax.ShapeDtypeStruct((N_NODES_K, D_FEAT_K), jnp.float32),
    )(partials)


@jax.jit
def kernel(x, index):
    idx2d = index.astype(jnp.int32).reshape(N_WINDOWS, WINDOW)
    partials = _sc_scatter_partials(x, idx2d)
    return _tc_combine(partials)
